# Initial kernel scaffold; baseline (speedup 1.0000x reference)
import jax
import jax.numpy as jnp
from jax.experimental import pallas as pl


def kernel(sent_logits, word_ids, labels, W_mlp):
    raise NotImplementedError("write your pallas kernel here")


# speedup vs baseline: 1.7800x; 1.7800x over previous
import functools

import jax
import jax.numpy as jnp
from jax import lax
from jax.experimental import pallas as pl
from jax.experimental.pallas import tpu as pltpu
from jax.experimental.pallas import tpu_sc as plsc

B, S, D = 16, 2048, 768
W_MAX = 1024
NL = 9
LANES = 16
SC_LANES = 10

NC, NS = 2, 16
CHUNK = 1024


def _probs_body(x_ref, w_ref, o_ref):
    x = x_ref[...]
    w = w_ref[...]
    lt = lax.dot_general(w, x, (((0,), (1,)), ((), ())),
                         preferred_element_type=jnp.float32,
                         precision=lax.Precision.HIGHEST)
    row = lax.broadcasted_iota(jnp.int32, lt.shape, 0)
    valid = row < NL
    m = jnp.max(jnp.where(valid, lt, -jnp.inf), axis=0, keepdims=True)
    e = jnp.where(valid, jnp.exp(lt - m), 0.0)
    p = e / jnp.sum(e, axis=0, keepdims=True)
    o_ref[0] = jnp.where(row == NL, 1.0, p)


def _probs_call(x2, w16):
    nchunk = S // CHUNK
    return pl.pallas_call(
        _probs_body,
        grid=(B * S // CHUNK,),
        in_specs=[
            pl.BlockSpec((CHUNK, D), lambda i: (i, 0)),
            pl.BlockSpec((D, LANES), lambda i: (0, 0)),
        ],
        out_specs=pl.BlockSpec((1, LANES, CHUNK),
                               lambda i: (i // nchunk, 0, i % nchunk)),
        out_shape=jax.ShapeDtypeStruct((B, LANES, S), jnp.float32),
        compiler_params=pltpu.CompilerParams(
            dimension_semantics=("arbitrary",)),
    )(x2, w16)


def _seg_body(probsT_hbm, wids_hbm, out_hbm, idx_v, pt_v, acc_v):
    c = lax.axis_index("c")
    s = lax.axis_index("s")
    b = c * NS + s

    @pl.when(b < B)
    def _():
        pltpu.sync_copy(wids_hbm.at[b], idx_v)
        pltpu.sync_copy(probsT_hbm.at[b], pt_v)

        def zero(i, carry):
            acc_v[i] = jnp.zeros((LANES,), jnp.float32)
            return carry
        lax.fori_loop(0, W_MAX, zero, 0)

        def chunk(t, carry):
            iw = idx_v[pl.ds(t * 16, 16)]
            for l in range(SC_LANES):
                vals = pt_v[l, pl.ds(t * 16, 16)]
                plsc.addupdate_scatter(
                    acc_v, [iw, jnp.full((16,), l, jnp.int32)], vals)
            return carry
        lax.fori_loop(0, S // 16, chunk, 0)

        pltpu.sync_copy(acc_v, out_hbm.at[b])


@functools.cache
def _seg_call():
    return pl.kernel(
        _seg_body,
        out_type=jax.ShapeDtypeStruct((B, W_MAX, LANES), jnp.float32),
        mesh=plsc.VectorSubcoreMesh(core_axis_name="c", subcore_axis_name="s",
                                    num_cores=NC, num_subcores=NS),
        compiler_params=pltpu.CompilerParams(needs_layout_passes=False,
                                             use_tc_tiling_on_sc=False),
        scratch_types=[
            pltpu.VMEM((S,), jnp.int32),
            pltpu.VMEM((LANES, S), jnp.float32),
            pltpu.VMEM((W_MAX, LANES), jnp.float32),
        ],
    )


def _final_body(sums_ref, labels_ref, la_ref, loss_ref):
    sums = sums_ref[...]
    lane = lax.broadcasted_iota(jnp.int32, sums.shape, 2)
    cnt = jnp.sum(jnp.where(lane == NL, sums, 0.0), axis=-1, keepdims=True)
    avg = sums / jnp.maximum(cnt, 1.0)
    la_ref[...] = avg[..., :NL]
    valid = lane < NL
    e = jnp.where(valid, jnp.exp(avg), 0.0)
    lse = jnp.log(jnp.sum(e, axis=-1))
    lab = labels_ref[...][..., None]
    picked = jnp.sum(jnp.where(lane == lab, avg, 0.0), axis=-1)
    loss_ref[0, 0] = jnp.sum(lse - picked) / float(B * W_MAX)


def _final_call(sums3, labels):
    return pl.pallas_call(
        _final_body,
        in_specs=[
            pl.BlockSpec((B, W_MAX, LANES), lambda: (0, 0, 0)),
            pl.BlockSpec((B, W_MAX), lambda: (0, 0)),
        ],
        out_specs=[
            pl.BlockSpec((B, W_MAX, NL), lambda: (0, 0, 0)),
            pl.BlockSpec(memory_space=pltpu.SMEM),
        ],
        out_shape=[
            jax.ShapeDtypeStruct((B, W_MAX, NL), jnp.float32),
            jax.ShapeDtypeStruct((1, 1), jnp.float32),
        ],
    )(sums3, labels)


def kernel(sent_logits, word_ids, labels, W_mlp):
    x2 = sent_logits.reshape(B * S, D)
    w16 = jnp.zeros((D, LANES), jnp.float32).at[:, :NL].set(W_mlp)
    probs_t = _probs_call(x2, w16)
    sums = _seg_call()(probs_t, word_ids)
    la, loss = _final_call(sums, labels)
    return la, loss.reshape(())


# speedup vs baseline: 2.7023x; 1.5181x over previous
import functools

import jax
import jax.numpy as jnp
from jax import lax
from jax.experimental import pallas as pl
from jax.experimental.pallas import tpu as pltpu
from jax.experimental.pallas import tpu_sc as plsc

B, S, D = 16, 2048, 768
W_MAX = 1024
NL = 9
LANES = 16
SC_LANES = 10

NC, NS = 2, 16
CHUNK = 2048


def _probs_body(x_ref, w_ref, o_ref):
    x = x_ref[...]
    w = w_ref[...]
    lt = lax.dot_general(w, x, (((0,), (1,)), ((), ())),
                         preferred_element_type=jnp.float32,
                         precision=lax.Precision.DEFAULT)
    row = lax.broadcasted_iota(jnp.int32, lt.shape, 0)
    valid = row < NL
    m = jnp.max(jnp.where(valid, lt, -jnp.inf), axis=0, keepdims=True)
    e = jnp.where(valid, jnp.exp(lt - m), 0.0)
    p = e / jnp.sum(e, axis=0, keepdims=True)
    o_ref[0] = jnp.where(row == NL, 1.0, p)


def _probs_call(x2, w16):
    nchunk = S // CHUNK
    return pl.pallas_call(
        _probs_body,
        grid=(B * S // CHUNK,),
        in_specs=[
            pl.BlockSpec((CHUNK, D), lambda i: (i, 0)),
            pl.BlockSpec((D, LANES), lambda i: (0, 0)),
        ],
        out_specs=pl.BlockSpec((1, LANES, CHUNK),
                               lambda i: (i // nchunk, 0, i % nchunk)),
        out_shape=jax.ShapeDtypeStruct((B, LANES, S), jnp.float32),
        compiler_params=pltpu.CompilerParams(
            dimension_semantics=("arbitrary",)),
    )(x2, w16)


def _seg_body(probsT_hbm, wids_hbm, out_hbm, idx_v, pt_v, acc_v):
    c = lax.axis_index("c")
    s = lax.axis_index("s")
    b = c * NS + s

    @pl.when(b < B)
    def _():
        pltpu.sync_copy(wids_hbm.at[b], idx_v)
        pltpu.sync_copy(probsT_hbm.at[b], pt_v)

        def zero(i, carry):
            acc_v[i] = jnp.zeros((LANES,), jnp.float32)
            return carry
        lax.fori_loop(0, W_MAX, zero, 0)

        def chunk(t, carry):
            iw = idx_v[pl.ds(t * 16, 16)]
            for l in range(SC_LANES):
                vals = pt_v[l, pl.ds(t * 16, 16)]
                plsc.addupdate_scatter(
                    acc_v, [iw, jnp.full((16,), l, jnp.int32)], vals)
            return carry
        lax.fori_loop(0, S // 16, chunk, 0)

        pltpu.sync_copy(acc_v, out_hbm.at[b])


@functools.cache
def _seg_call():
    return pl.kernel(
        _seg_body,
        out_type=jax.ShapeDtypeStruct((B, W_MAX, LANES), jnp.float32),
        mesh=plsc.VectorSubcoreMesh(core_axis_name="c", subcore_axis_name="s",
                                    num_cores=NC, num_subcores=NS),
        compiler_params=pltpu.CompilerParams(needs_layout_passes=False,
                                             use_tc_tiling_on_sc=False),
        scratch_types=[
            pltpu.VMEM((S,), jnp.int32),
            pltpu.VMEM((LANES, S), jnp.float32),
            pltpu.VMEM((W_MAX, LANES), jnp.float32),
        ],
    )


def _final_body(sums_ref, labels_ref, la_ref, loss_ref):
    sums = sums_ref[...]
    lane = lax.broadcasted_iota(jnp.int32, sums.shape, 2)
    cnt = jnp.sum(jnp.where(lane == NL, sums, 0.0), axis=-1, keepdims=True)
    avg = sums / jnp.maximum(cnt, 1.0)
    la_ref[...] = avg[..., :NL]
    valid = lane < NL
    e = jnp.where(valid, jnp.exp(avg), 0.0)
    lse = jnp.log(jnp.sum(e, axis=-1))
    lab = labels_ref[...][..., None]
    picked = jnp.sum(jnp.where(lane == lab, avg, 0.0), axis=-1)
    loss_ref[0, 0] = jnp.sum(lse - picked) / float(B * W_MAX)


def _final_call(sums3, labels):
    return pl.pallas_call(
        _final_body,
        in_specs=[
            pl.BlockSpec((B, W_MAX, LANES), lambda: (0, 0, 0)),
            pl.BlockSpec((B, W_MAX), lambda: (0, 0)),
        ],
        out_specs=[
            pl.BlockSpec((B, W_MAX, NL), lambda: (0, 0, 0)),
            pl.BlockSpec(memory_space=pltpu.SMEM),
        ],
        out_shape=[
            jax.ShapeDtypeStruct((B, W_MAX, NL), jnp.float32),
            jax.ShapeDtypeStruct((1, 1), jnp.float32),
        ],
    )(sums3, labels)


def kernel(sent_logits, word_ids, labels, W_mlp):
    x2 = sent_logits.reshape(B * S, D)
    w16 = jnp.zeros((D, LANES), jnp.float32).at[:, :NL].set(W_mlp)
    probs_t = _probs_call(x2, w16)
    sums = _seg_call()(probs_t, word_ids)
    la, loss = _final_call(sums, labels)
    return la, loss.reshape(())


# speedup vs baseline: 2.7449x; 1.0158x over previous
import functools

import jax
import jax.numpy as jnp
from jax import lax
from jax.experimental import pallas as pl
from jax.experimental.pallas import tpu as pltpu
from jax.experimental.pallas import tpu_sc as plsc

B, S, D = 16, 2048, 768
W_MAX = 1024
NL = 9
LANES = 16
SC_LANES = 10

NC, NS = 2, 16
CHUNK = 2048


def _probs_body(x_ref, w_ref, o_ref):
    x = x_ref[...]
    w = w_ref[...]
    lt = lax.dot_general(w, x, (((0,), (1,)), ((), ())),
                         preferred_element_type=jnp.float32,
                         precision=lax.Precision.DEFAULT)
    row = lax.broadcasted_iota(jnp.int32, lt.shape, 0)
    valid = row < NL
    m = jnp.max(jnp.where(valid, lt, -jnp.inf), axis=0, keepdims=True)
    e = jnp.where(valid, jnp.exp(lt - m), 0.0)
    p = e / jnp.sum(e, axis=0, keepdims=True)
    o_ref[0] = jnp.where(row == NL, 1.0, p)


def _probs_call(x2, w16):
    nchunk = S // CHUNK
    return pl.pallas_call(
        _probs_body,
        grid=(B * S // CHUNK,),
        in_specs=[
            pl.BlockSpec((CHUNK, D), lambda i: (i, 0)),
            pl.BlockSpec((D, LANES), lambda i: (0, 0)),
        ],
        out_specs=pl.BlockSpec((1, LANES, CHUNK),
                               lambda i: (i // nchunk, 0, i % nchunk)),
        out_shape=jax.ShapeDtypeStruct((B, LANES, S), jnp.float32),
        compiler_params=pltpu.CompilerParams(
            dimension_semantics=("arbitrary",)),
    )(x2, w16)


def _seg_body(probsT_hbm, wids_hbm, zeros_hbm, out_hbm, idx_v, pt_v, acc_v, sem):
    c = lax.axis_index("c")
    s = lax.axis_index("s")
    b = c * NS + s

    @pl.when(b < B)
    def _():
        cp1 = pltpu.async_copy(wids_hbm.at[b], idx_v, sem)
        cp2 = pltpu.async_copy(probsT_hbm.at[b], pt_v, sem)
        cp3 = pltpu.async_copy(zeros_hbm, acc_v, sem)
        cp1.wait()
        cp2.wait()
        cp3.wait()

        def chunk(t, carry):
            iw = idx_v[pl.ds(t * 16, 16)]
            for l in range(SC_LANES):
                vals = pt_v[l, pl.ds(t * 16, 16)]
                plsc.addupdate_scatter(
                    acc_v, [iw, jnp.full((16,), l, jnp.int32)], vals)
            return carry
        lax.fori_loop(0, S // 16, chunk, 0)

        pltpu.sync_copy(acc_v, out_hbm.at[b])


@functools.cache
def _seg_call():
    return pl.kernel(
        _seg_body,
        out_type=jax.ShapeDtypeStruct((B, W_MAX, LANES), jnp.float32),
        mesh=plsc.VectorSubcoreMesh(core_axis_name="c", subcore_axis_name="s",
                                    num_cores=NC, num_subcores=NS),
        compiler_params=pltpu.CompilerParams(needs_layout_passes=False,
                                             use_tc_tiling_on_sc=False),
        scratch_types=[
            pltpu.VMEM((S,), jnp.int32),
            pltpu.VMEM((LANES, S), jnp.float32),
            pltpu.VMEM((W_MAX, LANES), jnp.float32),
            pltpu.SemaphoreType.DMA,
        ],
    )


def _final_body(sums_ref, labels_ref, la_ref, loss_ref):
    sums = sums_ref[...]
    lane = lax.broadcasted_iota(jnp.int32, sums.shape, 2)
    cnt = jnp.sum(jnp.where(lane == NL, sums, 0.0), axis=-1, keepdims=True)
    avg = sums / jnp.maximum(cnt, 1.0)
    la_ref[...] = avg[..., :NL]
    valid = lane < NL
    e = jnp.where(valid, jnp.exp(avg), 0.0)
    lse = jnp.log(jnp.sum(e, axis=-1))
    lab = labels_ref[...][..., None]
    picked = jnp.sum(jnp.where(lane == lab, avg, 0.0), axis=-1)
    loss_ref[0, 0] = jnp.sum(lse - picked) / float(B * W_MAX)


def _final_call(sums3, labels):
    return pl.pallas_call(
        _final_body,
        in_specs=[
            pl.BlockSpec((B, W_MAX, LANES), lambda: (0, 0, 0)),
            pl.BlockSpec((B, W_MAX), lambda: (0, 0)),
        ],
        out_specs=[
            pl.BlockSpec((B, W_MAX, NL), lambda: (0, 0, 0)),
            pl.BlockSpec(memory_space=pltpu.SMEM),
        ],
        out_shape=[
            jax.ShapeDtypeStruct((B, W_MAX, NL), jnp.float32),
            jax.ShapeDtypeStruct((1, 1), jnp.float32),
        ],
    )(sums3, labels)


def kernel(sent_logits, word_ids, labels, W_mlp):
    x2 = sent_logits.reshape(B * S, D)
    w16 = jnp.zeros((D, LANES), jnp.float32).at[:, :NL].set(W_mlp)
    probs_t = _probs_call(x2, w16)
    zeros = jnp.zeros((W_MAX, LANES), jnp.float32)
    sums = _seg_call()(probs_t, word_ids, zeros)
    la, loss = _final_call(sums, labels)
    return la, loss.reshape(())


# speedup vs baseline: 2.7468x; 1.0007x over previous
import functools

import jax
import jax.numpy as jnp
from jax import lax
from jax.experimental import pallas as pl
from jax.experimental.pallas import tpu as pltpu
from jax.experimental.pallas import tpu_sc as plsc

B, S, D = 16, 2048, 768
W_MAX = 1024
NL = 9
LANES = 16
SC_LANES = 10

NC, NS = 2, 16
CHUNK = 2048


def _probs_body(x_ref, w_ref, o_ref):
    x = x_ref[...]
    w = w_ref[...]
    lt = lax.dot_general(w, x, (((0,), (1,)), ((), ())),
                         preferred_element_type=jnp.float32,
                         precision=lax.Precision.DEFAULT)
    row = lax.broadcasted_iota(jnp.int32, lt.shape, 0)
    valid = row < NL
    m = jnp.max(jnp.where(valid, lt, -jnp.inf), axis=0, keepdims=True)
    e = jnp.where(valid, jnp.exp(lt - m), 0.0)
    p = e / jnp.sum(e, axis=0, keepdims=True)
    o_ref[0] = jnp.where(row == NL, 1.0, p)


def _probs_call(x2, w16):
    nchunk = S // CHUNK
    return pl.pallas_call(
        _probs_body,
        grid=(B * S // CHUNK,),
        in_specs=[
            pl.BlockSpec((CHUNK, D), lambda i: (i, 0)),
            pl.BlockSpec((D, LANES), lambda i: (0, 0)),
        ],
        out_specs=pl.BlockSpec((1, LANES, CHUNK),
                               lambda i: (i // nchunk, 0, i % nchunk)),
        out_shape=jax.ShapeDtypeStruct((B, LANES, S), jnp.float32),
        compiler_params=pltpu.CompilerParams(
            dimension_semantics=("arbitrary",)),
    )(x2, w16)


def _seg_body(probsT_hbm, wids_hbm, zeros_hbm, out_hbm, idx_v, pt_v, acc_v, sem):
    c = lax.axis_index("c")
    s = lax.axis_index("s")
    b = c * NS + s

    @pl.when(b < B)
    def _():
        cp1 = pltpu.async_copy(wids_hbm.at[b], idx_v, sem)
        cp2 = pltpu.async_copy(probsT_hbm.at[b], pt_v, sem)
        cp3 = pltpu.async_copy(zeros_hbm, acc_v, sem)
        cp1.wait()
        cp2.wait()
        cp3.wait()

        def chunk(t, carry):
            for u in range(2):
                base = t * 32 + u * 16
                iw = idx_v[pl.ds(base, 16)]
                for l in range(SC_LANES):
                    vals = pt_v[l, pl.ds(base, 16)]
                    plsc.addupdate_scatter(
                        acc_v, [iw, jnp.full((16,), l, jnp.int32)], vals)
            return carry
        lax.fori_loop(0, S // 32, chunk, 0)

        pltpu.sync_copy(acc_v, out_hbm.at[b])


@functools.cache
def _seg_call():
    return pl.kernel(
        _seg_body,
        out_type=jax.ShapeDtypeStruct((B, W_MAX, LANES), jnp.float32),
        mesh=plsc.VectorSubcoreMesh(core_axis_name="c", subcore_axis_name="s",
                                    num_cores=NC, num_subcores=NS),
        compiler_params=pltpu.CompilerParams(needs_layout_passes=False,
                                             use_tc_tiling_on_sc=False),
        scratch_types=[
            pltpu.VMEM((S,), jnp.int32),
            pltpu.VMEM((LANES, S), jnp.float32),
            pltpu.VMEM((W_MAX, LANES), jnp.float32),
            pltpu.SemaphoreType.DMA,
        ],
    )


def _final_body(sums_ref, labels_ref, la_ref, loss_ref):
    sums = sums_ref[...]
    lane = lax.broadcasted_iota(jnp.int32, sums.shape, 2)
    cnt = jnp.sum(jnp.where(lane == NL, sums, 0.0), axis=-1, keepdims=True)
    avg = sums / jnp.maximum(cnt, 1.0)
    la_ref[...] = avg[..., :NL]
    valid = lane < NL
    e = jnp.where(valid, jnp.exp(avg), 0.0)
    lse = jnp.log(jnp.sum(e, axis=-1))
    lab = labels_ref[...][..., None]
    picked = jnp.sum(jnp.where(lane == lab, avg, 0.0), axis=-1)
    loss_ref[0, 0] = jnp.sum(lse - picked) / float(B * W_MAX)


def _final_call(sums3, labels):
    return pl.pallas_call(
        _final_body,
        in_specs=[
            pl.BlockSpec((B, W_MAX, LANES), lambda: (0, 0, 0)),
            pl.BlockSpec((B, W_MAX), lambda: (0, 0)),
        ],
        out_specs=[
            pl.BlockSpec((B, W_MAX, NL), lambda: (0, 0, 0)),
            pl.BlockSpec(memory_space=pltpu.SMEM),
        ],
        out_shape=[
            jax.ShapeDtypeStruct((B, W_MAX, NL), jnp.float32),
            jax.ShapeDtypeStruct((1, 1), jnp.float32),
        ],
    )(sums3, labels)


def kernel(sent_logits, word_ids, labels, W_mlp):
    x2 = sent_logits.reshape(B * S, D)
    w16 = jnp.zeros((D, LANES), jnp.float32).at[:, :NL].set(W_mlp)
    probs_t = _probs_call(x2, w16)
    zeros = jnp.zeros((W_MAX, LANES), jnp.float32)
    sums = _seg_call()(probs_t, word_ids, zeros)
    la, loss = _final_call(sums, labels)
    return la, loss.reshape(())


# speedup vs baseline: 3.3051x; 1.2033x over previous
import functools

import jax
import jax.numpy as jnp
from jax import lax
from jax.experimental import pallas as pl
from jax.experimental.pallas import tpu as pltpu
from jax.experimental.pallas import tpu_sc as plsc

B, S, D = 16, 2048, 768
W_MAX = 1024
NL = 9
LANES = 16
SC_LANES = 10

NC, NS = 2, 16
CHUNK = 2048


def _probs_body(x_ref, w_ref, o_ref):
    x = x_ref[0]
    w = w_ref[...]
    lt = lax.dot_general(w, x, (((0,), (1,)), ((), ())),
                         preferred_element_type=jnp.float32,
                         precision=lax.Precision.DEFAULT)
    m = jnp.max(lt, axis=0, keepdims=True)
    e = jnp.exp(lt - m)
    p = e / jnp.sum(e, axis=0, keepdims=True)
    o_ref[0, :NL, :] = p
    o_ref[0, NL:SC_LANES, :] = jnp.ones((1, CHUNK), jnp.float32)


def _probs_call(x3, w_mlp):
    return pl.pallas_call(
        _probs_body,
        grid=(B,),
        in_specs=[
            pl.BlockSpec((1, CHUNK, D), lambda i: (i, 0, 0)),
            pl.BlockSpec((D, NL), lambda i: (0, 0)),
        ],
        out_specs=pl.BlockSpec((1, LANES, CHUNK), lambda i: (i, 0, 0)),
        out_shape=jax.ShapeDtypeStruct((B, LANES, S), jnp.float32),
        compiler_params=pltpu.CompilerParams(
            dimension_semantics=("arbitrary",)),
    )(x3, w_mlp)


def _seg_body(probsT_hbm, wids_hbm, zeros_hbm, out_hbm, idx_v, pt_v, acc_v, sem):
    c = lax.axis_index("c")
    s = lax.axis_index("s")
    b = c * NS + s

    @pl.when(b < B)
    def _():
        cp1 = pltpu.async_copy(wids_hbm.at[b], idx_v, sem)
        cp2 = pltpu.async_copy(probsT_hbm.at[b], pt_v, sem)
        cp3 = pltpu.async_copy(zeros_hbm, acc_v, sem)
        cp1.wait()
        cp2.wait()
        cp3.wait()

        def chunk(t, carry):
            for u in range(2):
                base = t * 32 + u * 16
                iw = idx_v[pl.ds(base, 16)]
                for l in range(SC_LANES):
                    vals = pt_v[l, pl.ds(base, 16)]
                    plsc.addupdate_scatter(
                        acc_v, [iw + (l * W_MAX)], vals)
            return carry
        lax.fori_loop(0, S // 32, chunk, 0)

        pltpu.sync_copy(acc_v, out_hbm.at[b])


@functools.cache
def _seg_call():
    return pl.kernel(
        _seg_body,
        out_type=jax.ShapeDtypeStruct((B, SC_LANES * W_MAX), jnp.float32),
        mesh=plsc.VectorSubcoreMesh(core_axis_name="c", subcore_axis_name="s",
                                    num_cores=NC, num_subcores=NS),
        compiler_params=pltpu.CompilerParams(needs_layout_passes=False,
                                             use_tc_tiling_on_sc=False),
        scratch_types=[
            pltpu.VMEM((S,), jnp.int32),
            pltpu.VMEM((LANES, S), jnp.float32),
            pltpu.VMEM((SC_LANES * W_MAX,), jnp.float32),
            pltpu.SemaphoreType.DMA,
        ],
    )


def _final_body(sums_ref, labels_ref, la_ref, loss_ref):
    sums = sums_ref[...]
    lane = lax.broadcasted_iota(jnp.int32, sums.shape, 1)
    cnt = sums[:, NL:SC_LANES, :]
    avg = sums / jnp.maximum(cnt, 1.0)
    la_ref[...] = jnp.swapaxes(avg[:, :NL, :], 1, 2)
    valid = lane < NL
    e = jnp.where(valid, jnp.exp(avg), 0.0)
    lse = jnp.log(jnp.sum(e, axis=1))
    lab = labels_ref[...][:, None, :]
    picked = jnp.sum(jnp.where(lane == lab, avg, 0.0), axis=1)
    loss_ref[0, 0] = jnp.sum(lse - picked) / float(B * W_MAX)


def _final_call(sums3, labels):
    return pl.pallas_call(
        _final_body,
        in_specs=[
            pl.BlockSpec((B, SC_LANES, W_MAX), lambda: (0, 0, 0)),
            pl.BlockSpec((B, W_MAX), lambda: (0, 0)),
        ],
        out_specs=[
            pl.BlockSpec((B, W_MAX, NL), lambda: (0, 0, 0)),
            pl.BlockSpec(memory_space=pltpu.SMEM),
        ],
        out_shape=[
            jax.ShapeDtypeStruct((B, W_MAX, NL), jnp.float32),
            jax.ShapeDtypeStruct((1, 1), jnp.float32),
        ],
    )(sums3, labels)


def kernel(sent_logits, word_ids, labels, W_mlp):
    probs_t = _probs_call(sent_logits, W_mlp)
    zeros = jnp.zeros((SC_LANES * W_MAX,), jnp.float32)
    sums = _seg_call()(probs_t, word_ids, zeros)
    la, loss = _final_call(sums.reshape(B, SC_LANES, W_MAX), labels)
    return la, loss.reshape(())


# speedup vs baseline: 3.6879x; 1.1158x over previous
import functools

import jax
import jax.numpy as jnp
from jax import lax
from jax.experimental import pallas as pl
from jax.experimental.pallas import tpu as pltpu
from jax.experimental.pallas import tpu_sc as plsc

B, S, D = 16, 2048, 768
W_MAX = 1024
NL = 9
LANES = 16
SC_LANES = 10

NC, NS = 2, 16
CHUNK = 2048


def _probs_body(x_ref, w_ref, o_ref):
    x = x_ref[0]
    w = w_ref[...]
    lt = lax.dot_general(w, x, (((0,), (1,)), ((), ())),
                         preferred_element_type=jnp.float32,
                         precision=lax.Precision.DEFAULT)
    m = jnp.max(lt, axis=0, keepdims=True)
    e = jnp.exp(lt - m)
    p = e / jnp.sum(e, axis=0, keepdims=True)
    o_ref[0, :NL, :] = p
    o_ref[0, NL:SC_LANES, :] = jnp.ones((1, CHUNK), jnp.float32)


def _probs_call(x3, w_mlp):
    return pl.pallas_call(
        _probs_body,
        grid=(B,),
        in_specs=[
            pl.BlockSpec((1, CHUNK, D), lambda i: (i, 0, 0)),
            pl.BlockSpec((D, NL), lambda i: (0, 0)),
        ],
        out_specs=pl.BlockSpec((1, LANES, CHUNK), lambda i: (i, 0, 0)),
        out_shape=jax.ShapeDtypeStruct((B, LANES, S), jnp.float32),
        compiler_params=pltpu.CompilerParams(
            dimension_semantics=("arbitrary",)),
    )(x3, w_mlp)


def _seg_body(probsT_hbm, wids_hbm, zeros_hbm, out_hbm, idx_v, pt_v, acc_v, sem):
    c = lax.axis_index("c")
    s = lax.axis_index("s")
    b = c * NS + s

    @pl.when(b < B)
    def _():
        cp1 = pltpu.async_copy(wids_hbm.at[b], idx_v, sem)
        cp2 = pltpu.async_copy(probsT_hbm.at[b], pt_v, sem)
        cp3 = pltpu.async_copy(zeros_hbm, acc_v, sem)
        cp1.wait()
        cp2.wait()
        cp3.wait()

        def chunk(t, carry):
            for u in range(2):
                base = t * 32 + u * 16
                iw = idx_v[pl.ds(base, 16)]
                for l in range(SC_LANES):
                    vals = pt_v[l, pl.ds(base, 16)]
                    plsc.addupdate_scatter(
                        acc_v, [iw + (l * W_MAX)], vals)
            return carry
        lax.fori_loop(0, S // 32, chunk, 0)

        pltpu.sync_copy(acc_v, out_hbm.at[b])


@functools.cache
def _seg_call():
    return pl.kernel(
        _seg_body,
        out_type=jax.ShapeDtypeStruct((B, SC_LANES * W_MAX), jnp.float32),
        mesh=plsc.VectorSubcoreMesh(core_axis_name="c", subcore_axis_name="s",
                                    num_cores=NC, num_subcores=NS),
        compiler_params=pltpu.CompilerParams(needs_layout_passes=False,
                                             use_tc_tiling_on_sc=False),
        scratch_types=[
            pltpu.VMEM((S,), jnp.int32),
            pltpu.VMEM((LANES, S), jnp.float32),
            pltpu.VMEM((SC_LANES * W_MAX,), jnp.float32),
            pltpu.SemaphoreType.DMA,
        ],
    )


def _final_body(sums_ref, labels_ref, la_ref, loss_ref):
    sums = sums_ref[...]
    lane = lax.broadcasted_iota(jnp.int32, sums.shape, 1)
    cnt = sums[:, NL:SC_LANES, :]
    avg = sums / jnp.maximum(cnt, 1.0)
    la_ref[...] = avg[:, :NL, :]
    valid = lane < NL
    e = jnp.where(valid, jnp.exp(avg), 0.0)
    lse = jnp.log(jnp.sum(e, axis=1))
    lab = labels_ref[...][:, None, :]
    picked = jnp.sum(jnp.where(lane == lab, avg, 0.0), axis=1)
    loss_ref[0, 0] = jnp.sum(lse - picked) / float(B * W_MAX)


def _final_call(sums3, labels):
    return pl.pallas_call(
        _final_body,
        in_specs=[
            pl.BlockSpec((B, SC_LANES, W_MAX), lambda: (0, 0, 0)),
            pl.BlockSpec((B, W_MAX), lambda: (0, 0)),
        ],
        out_specs=[
            pl.BlockSpec((B, NL, W_MAX), lambda: (0, 0, 0)),
            pl.BlockSpec(memory_space=pltpu.SMEM),
        ],
        out_shape=[
            jax.ShapeDtypeStruct((B, NL, W_MAX), jnp.float32),
            jax.ShapeDtypeStruct((1, 1), jnp.float32),
        ],
    )(sums3, labels)


def kernel(sent_logits, word_ids, labels, W_mlp):
    probs_t = _probs_call(sent_logits, W_mlp)
    zeros = jnp.zeros((SC_LANES * W_MAX,), jnp.float32)
    sums = _seg_call()(probs_t, word_ids, zeros)
    la_t, loss = _final_call(sums.reshape(B, SC_LANES, W_MAX), labels)
    return jnp.swapaxes(la_t, 1, 2), loss.reshape(())
